# baseline (speedup 1.0000x reference)
import functools

import jax
import jax.numpy as jnp
from jax import lax
from jax.experimental import pallas as pl
from jax.experimental.pallas import tpu as pltpu
from jax.experimental.pallas import tpu_sc as plsc

B = 4096
F = 26
V = 100000
NC = 2
NS = 16
NW = NC * NS
RPW = B // NW
L = 16


def _lr_body(xw_hbm, w_hbm, bias_hbm, out_hbm, x_v, gath_v, acc_v, bias_v, sem):
    wid = lax.axis_index("s") * NC + lax.axis_index("c")
    base = wid * RPW

    pltpu.sync_copy(xw_hbm.at[wid], x_v)
    pltpu.sync_copy(bias_hbm, bias_v)

    for f in range(1, F):
        for c in range(RPW // L):
            sl = pl.ds(c * L, L)
            x_v[f, sl] = x_v[f, sl] + (f * V)

    copies = [
        pltpu.async_copy(w_hbm.at[x_v.at[f]], gath_v.at[f], sem)
        for f in range(F)
    ]
    for cp in copies:
        cp.wait()

    for c in range(RPW // L):
        sl = pl.ds(c * L, L)
        acc = bias_v[...]
        for f in range(F):
            acc = acc + gath_v[f, sl]
        acc_v[sl] = 1.0 / (1.0 + jnp.exp(-acc))

    pltpu.sync_copy(acc_v, out_hbm.at[pl.ds(base, RPW)])


@jax.jit
def _lr_sc(xw, wflat, bias16):
    call = functools.partial(
        pl.kernel,
        out_type=jax.ShapeDtypeStruct((B,), jnp.float32),
        mesh=plsc.VectorSubcoreMesh(core_axis_name="c", subcore_axis_name="s"),
        scratch_types=[
            pltpu.VMEM((F, RPW), jnp.int32),
            pltpu.VMEM((F, RPW), jnp.float32),
            pltpu.VMEM((RPW,), jnp.float32),
            pltpu.VMEM((L,), jnp.float32),
            pltpu.SemaphoreType.DMA,
        ],
    )(_lr_body)
    return call(xw, wflat, bias16)


def kernel(X, W, bias):
    xw = X.T.reshape(F, NW, RPW).transpose(1, 0, 2)
    wflat = W.reshape(F * V)
    bias16 = jnp.broadcast_to(bias, (L,))
    out = _lr_sc(xw, wflat, bias16)
    return out.reshape(B, 1)
